# baseline (speedup 1.0000x reference)
import functools

import jax
import jax.numpy as jnp
from jax import lax
from jax.experimental import pallas as pl
from jax.experimental.pallas import tpu as pltpu
from jax.experimental.pallas import tpu_sc as plsc

NB_EDGE_ALIGN = 2048
FC = 128
D = 1024


def _segsum_sc(src2d, pk, npad):
    nbt = pk.shape[1]
    nb = nbt // 16
    slab = npad // 16
    nsl = slab // 128
    mesh = plsc.VectorSubcoreMesh(core_axis_name="c", subcore_axis_name="s")

    @functools.partial(
        pl.kernel,
        out_type=jax.ShapeDtypeStruct((npad, D), jnp.float32),
        mesh=mesh,
        scratch_types=[
            pltpu.VMEM((2, 128), jnp.int32),
            pltpu.VMEM((128, FC), jnp.float32),
            pltpu.VMEM((128, FC), jnp.float32),
            pltpu.VMEM_SHARED((npad, FC), jnp.float32),
            pltpu.SemaphoreType.DMA,
        ],
    )
    def seg_kernel(src_hbm, pk_hbm, zero_hbm, out_hbm,
                   pk_v, rows_v, z_v, acc_sh, sem):
        cid = lax.axis_index("c")
        sid = lax.axis_index("s")
        pltpu.sync_copy(zero_hbm, z_v)
        row_base = sid * slab
        for cc in range(4):
            chunk = cid * 4 + cc

            @pl.loop(0, nsl)
            def _zero(i):
                pltpu.sync_copy(z_v, acc_sh.at[pl.ds(row_base + i * 128, 128)])

            plsc.subcore_barrier()

            @pl.loop(0, nb)
            def _edges(b):
                bi = sid * nb + b
                pltpu.sync_copy(pk_hbm.at[chunk, bi], pk_v)
                pltpu.async_copy(src_hbm.at[pk_v.at[0]], rows_v, sem).wait()
                pltpu.sync_copy(rows_v, acc_sh.at[pk_v.at[1]], add=True)

            plsc.subcore_barrier()

            pltpu.sync_copy(
                acc_sh.at[pl.ds(row_base, slab)],
                out_hbm.at[pl.ds(row_base, slab), pl.ds(chunk * FC, FC)])

    zeros = jnp.zeros((128, FC), jnp.float32)
    return seg_kernel(src2d, pk, zeros)


def _mm_bias(x, w, b, relu_in=False, bm=1024):
    m, k = x.shape
    n = w.shape[1]

    def body(x_ref, w_ref, b_ref, o_ref):
        xv = x_ref[...]
        if relu_in:
            xv = jnp.maximum(xv, 0.0)
        o_ref[...] = jnp.dot(
            xv, w_ref[...], preferred_element_type=jnp.float32) + b_ref[...]

    return pl.pallas_call(
        body,
        grid=(m // bm,),
        in_specs=[
            pl.BlockSpec((bm, k), lambda i: (i, 0)),
            pl.BlockSpec((k, n), lambda i: (0, 0)),
            pl.BlockSpec((1, n), lambda i: (0, 0)),
        ],
        out_specs=pl.BlockSpec((bm, n), lambda i: (i, 0)),
        out_shape=jax.ShapeDtypeStruct((m, n), jnp.float32),
    )(x, w, b.reshape(1, n))


def _ln_relu(x, scale, bias, bm=2048):
    m, d = x.shape

    def body(x_ref, s_ref, b_ref, o_ref):
        v = jnp.maximum(x_ref[...], 0.0)
        mu = jnp.mean(v, axis=-1, keepdims=True)
        var = jnp.mean((v - mu) ** 2, axis=-1, keepdims=True)
        o_ref[...] = (v - mu) / jnp.sqrt(var + 1e-5) * s_ref[...] + b_ref[...]

    return pl.pallas_call(
        body,
        grid=(m // bm,),
        in_specs=[
            pl.BlockSpec((bm, d), lambda i: (i, 0)),
            pl.BlockSpec((1, d), lambda i: (0, 0)),
            pl.BlockSpec((1, d), lambda i: (0, 0)),
        ],
        out_specs=pl.BlockSpec((bm, d), lambda i: (i, 0)),
        out_shape=jax.ShapeDtypeStruct((m, d), jnp.float32),
    )(x, scale.reshape(1, d), bias.reshape(1, d))


def _relu(x, bm=2048):
    m, d = x.shape

    def body(x_ref, o_ref):
        o_ref[...] = jnp.maximum(x_ref[...], 0.0)

    return pl.pallas_call(
        body,
        grid=(m // bm,),
        in_specs=[pl.BlockSpec((bm, d), lambda i: (i, 0))],
        out_specs=pl.BlockSpec((bm, d), lambda i: (i, 0)),
        out_shape=jax.ShapeDtypeStruct((m, d), jnp.float32),
    )(x)


def _tie(embs):
    n = embs.shape[0] // 2
    v = embs.shape[1]
    y = embs.reshape(n, 2, v)
    pos = y[:, 0, :]
    neg = y[:, 1, :]
    cp = jnp.concatenate([pos, neg], axis=1)
    cn = jnp.concatenate([neg, pos], axis=1)
    return jnp.stack([cp, cn], axis=1).reshape(2 * n, 2 * v)


def kernel(vlabels, clabels, edge_index, Wlc0, blc0, Wcl0, bcl0,
           Wlc1, blc1, Wcl1, bcl1, ln0_scale, ln0_bias, ln1_scale, ln1_bias):
    n_lit = vlabels.shape[0]
    n_cls = clabels.shape[0]
    e = edge_index.shape[1]
    npad = ((max(n_lit, n_cls) + NB_EDGE_ALIGN - 1)
            // NB_EDGE_ALIGN) * NB_EDGE_ALIGN

    lit = edge_index[0]
    cls = edge_index[1]
    ep = ((e + NB_EDGE_ALIGN - 1) // NB_EDGE_ALIGN) * NB_EDGE_ALIGN
    pad = ep - e
    zpad = jnp.zeros((pad,), jnp.int32)
    gpad = jnp.full((pad,), n_lit, jnp.int32)
    lit_g = jnp.concatenate([lit, zpad])
    cls_g = jnp.concatenate([cls, zpad])
    lit_d = jnp.concatenate([lit, gpad])
    cls_d = jnp.concatenate([cls, gpad])
    chunks = jnp.arange(8, dtype=jnp.int32)[:, None]
    gidx_l = lit_g[None, :] * 8 + chunks
    gidx_c = cls_g[None, :] * 8 + chunks
    nbt = ep // 128

    def _pack(g, d):
        gb = g.reshape(8, nbt, 1, 128)
        db = jnp.broadcast_to(d.reshape(1, nbt, 1, 128), (8, nbt, 1, 128))
        return jnp.concatenate([gb, db], axis=2)

    pk_lc = _pack(gidx_l, cls_d)
    pk_cl = _pack(gidx_c, lit_d)

    vlab_p = jnp.pad(vlabels, ((0, npad - n_lit), (0, 0)))

    proj0 = _mm_bias(vlab_p, Wlc0, blc0)
    cembs0 = _segsum_sc(proj0.reshape(-1, FC), pk_lc, npad)
    back0 = _mm_bias(cembs0, Wcl0, bcl0, relu_in=True)
    vpre0 = _segsum_sc(back0.reshape(-1, FC), pk_cl, npad)
    feat = _tie(_ln_relu(vpre0, ln0_scale, ln0_bias))

    proj1 = _mm_bias(feat, Wlc1, blc1)
    cembs1 = _segsum_sc(proj1.reshape(-1, FC), pk_lc, npad)
    cembs1_r = _relu(cembs1)
    back1 = _mm_bias(cembs1_r, Wcl1, bcl1)
    vpre1 = _segsum_sc(back1.reshape(-1, FC), pk_cl, npad)
    feat1 = _tie(_ln_relu(vpre1, ln1_scale, ln1_bias))

    vembs = jnp.concatenate([feat1[:n_lit], vlabels], axis=1)
    cembs_out = jnp.concatenate([cembs1_r[:n_cls], clabels], axis=1)
    return (vembs, cembs_out)


# speedup vs baseline: 1.6050x; 1.6050x over previous
import functools

import jax
import jax.numpy as jnp
from jax import lax
from jax.experimental import pallas as pl
from jax.experimental.pallas import tpu as pltpu
from jax.experimental.pallas import tpu_sc as plsc

NB_EDGE_ALIGN = 4096
FC = 128
D = 1024


def _segsum_sc(src2d, pk, npad):
    nbt = pk.shape[1]
    nb = nbt // 16
    slab = npad // 16
    nsl = slab // 128
    mesh = plsc.VectorSubcoreMesh(core_axis_name="c", subcore_axis_name="s")

    @functools.partial(
        pl.kernel,
        out_type=jax.ShapeDtypeStruct((npad, D), jnp.float32),
        mesh=mesh,
        scratch_types=[
            pltpu.VMEM((4, 128), jnp.int32),
            pltpu.VMEM((128, FC), jnp.float32),
            pltpu.VMEM((128, FC), jnp.float32),
            pltpu.VMEM_SHARED((npad, FC), jnp.float32),
            pltpu.SemaphoreType.DMA,
        ],
    )
    def seg_kernel(src_hbm, pk_hbm, zero_hbm, out_hbm,
                   pk_v, rows_v, z_v, acc_sh, sem):
        cid = lax.axis_index("c")
        sid = lax.axis_index("s")
        pltpu.sync_copy(zero_hbm, z_v)
        row_base = sid * slab
        for cc in range(4):
            chunk = cid * 4 + cc

            @pl.loop(0, nsl)
            def _zero(i):
                pltpu.sync_copy(z_v, acc_sh.at[pl.ds(row_base + i * 128, 128)])

            plsc.subcore_barrier()

            @pl.loop(0, nb // 2)
            def _edges(b):
                bi = sid * (nb // 2) + b
                pltpu.sync_copy(pk_hbm.at[chunk, bi], pk_v)
                pltpu.async_copy(src_hbm.at[pk_v.at[0]], rows_v, sem).wait()
                pltpu.sync_copy(rows_v, acc_sh.at[pk_v.at[1]], add=True)
                pltpu.async_copy(src_hbm.at[pk_v.at[2]], rows_v, sem).wait()
                pltpu.sync_copy(rows_v, acc_sh.at[pk_v.at[3]], add=True)

            plsc.subcore_barrier()

            @pl.loop(0, nsl)
            def _drain(i):
                r0 = row_base + i * 128
                pltpu.sync_copy(acc_sh.at[pl.ds(r0, 128)], rows_v)
                pltpu.sync_copy(
                    rows_v,
                    out_hbm.at[pl.ds(r0, 128), pl.ds(chunk * FC, FC)])

    zeros = jnp.zeros((128, FC), jnp.float32)
    return seg_kernel(src2d, pk, zeros)


def _mm_bias(x, w, b, relu_in=False, bm=1024):
    m, k = x.shape
    n = w.shape[1]

    def body(x_ref, w_ref, b_ref, o_ref):
        xv = x_ref[...]
        if relu_in:
            xv = jnp.maximum(xv, 0.0)
        o_ref[...] = jnp.dot(
            xv, w_ref[...], preferred_element_type=jnp.float32) + b_ref[...]

    return pl.pallas_call(
        body,
        grid=(m // bm,),
        in_specs=[
            pl.BlockSpec((bm, k), lambda i: (i, 0)),
            pl.BlockSpec((k, n), lambda i: (0, 0)),
            pl.BlockSpec((1, n), lambda i: (0, 0)),
        ],
        out_specs=pl.BlockSpec((bm, n), lambda i: (i, 0)),
        out_shape=jax.ShapeDtypeStruct((m, n), jnp.float32),
    )(x, w, b.reshape(1, n))


def _ln_relu(x, scale, bias, bm=2048):
    m, d = x.shape

    def body(x_ref, s_ref, b_ref, o_ref):
        v = jnp.maximum(x_ref[...], 0.0)
        mu = jnp.mean(v, axis=-1, keepdims=True)
        var = jnp.mean((v - mu) ** 2, axis=-1, keepdims=True)
        o_ref[...] = (v - mu) / jnp.sqrt(var + 1e-5) * s_ref[...] + b_ref[...]

    return pl.pallas_call(
        body,
        grid=(m // bm,),
        in_specs=[
            pl.BlockSpec((bm, d), lambda i: (i, 0)),
            pl.BlockSpec((1, d), lambda i: (0, 0)),
            pl.BlockSpec((1, d), lambda i: (0, 0)),
        ],
        out_specs=pl.BlockSpec((bm, d), lambda i: (i, 0)),
        out_shape=jax.ShapeDtypeStruct((m, d), jnp.float32),
    )(x, scale.reshape(1, d), bias.reshape(1, d))


def _relu(x, bm=2048):
    m, d = x.shape

    def body(x_ref, o_ref):
        o_ref[...] = jnp.maximum(x_ref[...], 0.0)

    return pl.pallas_call(
        body,
        grid=(m // bm,),
        in_specs=[pl.BlockSpec((bm, d), lambda i: (i, 0))],
        out_specs=pl.BlockSpec((bm, d), lambda i: (i, 0)),
        out_shape=jax.ShapeDtypeStruct((m, d), jnp.float32),
    )(x)


def _tie(embs):
    n = embs.shape[0] // 2
    v = embs.shape[1]
    y = embs.reshape(n, 2, v)
    pos = y[:, 0, :]
    neg = y[:, 1, :]
    cp = jnp.concatenate([pos, neg], axis=1)
    cn = jnp.concatenate([neg, pos], axis=1)
    return jnp.stack([cp, cn], axis=1).reshape(2 * n, 2 * v)


def kernel(vlabels, clabels, edge_index, Wlc0, blc0, Wcl0, bcl0,
           Wlc1, blc1, Wcl1, bcl1, ln0_scale, ln0_bias, ln1_scale, ln1_bias):
    n_lit = vlabels.shape[0]
    n_cls = clabels.shape[0]
    e = edge_index.shape[1]
    npad = ((max(n_lit, n_cls) + 2048) // 2048) * 2048

    lit = edge_index[0]
    cls = edge_index[1]
    ep = ((e + NB_EDGE_ALIGN - 1) // NB_EDGE_ALIGN) * NB_EDGE_ALIGN
    pad = ep - e
    zpad = jnp.zeros((pad,), jnp.int32)
    gpad = jnp.full((pad,), n_lit, jnp.int32)
    lit_g = jnp.concatenate([lit, zpad])
    cls_g = jnp.concatenate([cls, zpad])
    lit_d = jnp.concatenate([lit, gpad])
    cls_d = jnp.concatenate([cls, gpad])
    chunks = jnp.arange(8, dtype=jnp.int32)[:, None]
    gidx_l = lit_g[None, :] * 8 + chunks
    gidx_c = cls_g[None, :] * 8 + chunks
    nbt = ep // 128

    def _pack(g, d):
        gb = g.reshape(8, nbt, 1, 128)
        db = jnp.broadcast_to(d.reshape(1, nbt, 1, 128), (8, nbt, 1, 128))
        pairs = jnp.concatenate([gb, db], axis=2)
        return pairs.reshape(8, nbt // 2, 4, 128)

    pk_lc = _pack(gidx_l, cls_d)
    pk_cl = _pack(gidx_c, lit_d)

    vlab_p = jnp.pad(vlabels, ((0, npad - n_lit), (0, 0)))

    proj0 = _mm_bias(vlab_p, Wlc0, blc0)
    cembs0 = _segsum_sc(proj0.reshape(-1, FC), pk_lc, npad)
    back0 = _mm_bias(cembs0, Wcl0, bcl0, relu_in=True)
    vpre0 = _segsum_sc(back0.reshape(-1, FC), pk_cl, npad)
    feat = _tie(_ln_relu(vpre0, ln0_scale, ln0_bias))

    proj1 = _mm_bias(feat, Wlc1, blc1)
    cembs1 = _segsum_sc(proj1.reshape(-1, FC), pk_lc, npad)
    cembs1_r = _relu(cembs1)
    back1 = _mm_bias(cembs1_r, Wcl1, bcl1)
    vpre1 = _segsum_sc(back1.reshape(-1, FC), pk_cl, npad)
    feat1 = _tie(_ln_relu(vpre1, ln1_scale, ln1_bias))

    vembs = jnp.concatenate([feat1[:n_lit], vlabels], axis=1)
    cembs_out = jnp.concatenate([cembs1_r[:n_cls], clabels], axis=1)
    return (vembs, cembs_out)
